# speedup vs baseline: 11.3788x; 11.3788x over previous
import jax
import jax.numpy as jnp
from jax.experimental import pallas as pl

_THRESHOLD = 0.5
_NEG_POS_RATIO = 3
_ALPHA = 10.0

_PB = 1024


def _match_kernel(gb_ref, gl_ref, caT_ref, plT_ref, tc_ref, pos_ref, loc_ref):
    O = gb_ref.shape[1]
    P = caT_ref.shape[1]
    f32 = jnp.float32

    gb = gb_ref[0]
    bx1 = gb[:, 0:1]
    by1 = gb[:, 1:2]
    bx2 = gb[:, 2:3]
    by2 = gb[:, 3:4]
    ca = caT_ref[...]
    acx = ca[0:1, :]
    acy = ca[1:2, :]
    aw = ca[2:3, :]
    ah = ca[3:4, :]
    ax1 = acx - aw / 2.0
    ay1 = acy - ah / 2.0
    ax2 = acx + aw / 2.0
    ay2 = acy + ah / 2.0
    a2 = (ax2 - ax1) * (ay2 - ay1)
    a1 = (bx2 - bx1) * (by2 - by1)

    lox = jnp.maximum(bx1, ax1)
    loy = jnp.maximum(by1, ay1)
    hix = jnp.minimum(bx2, ax2)
    hiy = jnp.minimum(by2, ay2)
    w = jnp.maximum(hix - lox, 0.0)
    h = jnp.maximum(hiy - loy, 0.0)
    inter = w * h
    iou = inter / (a1 + a2 - inter)

    iota_p = jax.lax.broadcasted_iota(jnp.int32, (O, P), 1)
    rmax = jnp.max(iou, axis=1, keepdims=True)
    ridx = jnp.min(jnp.where(iou == rmax, iota_p, P), axis=1, keepdims=True)
    iou_mod = jnp.where(iota_p == ridx, 1.0, iou)

    cmax = jnp.max(iou_mod, axis=0, keepdims=True)
    posf = (cmax >= _THRESHOLD).astype(f32)
    iota_o = jax.lax.broadcasted_iota(jnp.int32, (O, P), 0)
    bo = jnp.min(jnp.where(iou_mod == cmax, iota_o, O), axis=0, keepdims=True)
    onehot = (iota_o == bo).astype(f32)

    lab = gl_ref[0].astype(f32)
    tcls = jnp.sum(onehot * (lab + 1.0), axis=0, keepdims=True)
    tc_ref[...] = (tcls * posf).astype(jnp.int32).reshape(1, 1, P)
    pos_ref[...] = posf.reshape(1, 1, P)

    gx1 = jnp.sum(onehot * bx1, axis=0, keepdims=True)
    gy1 = jnp.sum(onehot * by1, axis=0, keepdims=True)
    gx2 = jnp.sum(onehot * bx2, axis=0, keepdims=True)
    gy2 = jnp.sum(onehot * by2, axis=0, keepdims=True)
    gcx = (gx1 + gx2) / 2.0
    gcy = (gy1 + gy2) / 2.0
    gw = gx2 - gx1
    gh = gy2 - gy1
    t0 = (gcx - acx) / (aw / 10.0)
    t1 = (gcy - acy) / (ah / 10.0)
    t2 = jnp.log(gw / aw) * 5.0
    t3 = jnp.log(gh / ah) * 5.0

    pq = plT_ref[0]
    s = (jnp.abs(pq[0:1, :] - t0) + jnp.abs(pq[1:2, :] - t1)
         + jnp.abs(pq[2:3, :] - t2) + jnp.abs(pq[3:4, :] - t3))
    loc_ref[...] = jnp.sum(s * posf).reshape(1, 1, 1)


def _ce_kernel(x_ref, tc_ref, out_ref):
    PB, C = x_ref.shape[1], x_ref.shape[2]
    x = x_ref[0]
    ones = jnp.ones((C, 1), jnp.float32)
    e = jnp.exp(x)
    s = jax.lax.dot_general(e, ones, (((1,), (0,)), ((), ())),
                            preferred_element_type=jnp.float32)
    tc = tc_ref[0]
    iota_c = jax.lax.broadcasted_iota(jnp.int32, (PB, C), 1)
    xm = jnp.where(iota_c == tc, x, 0.0)
    xt = jax.lax.dot_general(xm, ones, (((1,), (0,)), ((), ())),
                             preferred_element_type=jnp.float32)
    out_ref[0] = jnp.log(s) - xt


def _mine_kernel(conf_ref, pos_ref, loc_ref, tot_ref, cl_ref, ll_ref):
    f32 = jnp.float32
    i32 = jnp.int32
    conf = conf_ref[...]
    pos = pos_ref[...]
    npos = jnp.sum(pos, axis=1, keepdims=True)
    K = (npos * float(_NEG_POS_RATIO)).astype(i32)
    v = conf * (1.0 - pos)
    u = jax.lax.bitcast_convert_type(v, i32)

    T = jnp.zeros(K.shape, i32)
    for b in range(30, -1, -1):
        cand = T | jnp.int32(1 << b)
        cnt = jnp.sum((u >= cand).astype(i32), axis=1, keepdims=True)
        T = jnp.where(cnt >= K, cand, T)
    Tf = jax.lax.bitcast_convert_type(T, f32)

    gt = v > Tf
    cntgt = jnp.sum(gt.astype(f32), axis=1, keepdims=True)
    sumgt = jnp.sum(jnp.where(gt, v, 0.0), axis=1, keepdims=True)
    neg = sumgt + (K.astype(f32) - cntgt) * Tf

    pos_sum = jnp.sum(conf * pos)
    npt = jnp.sum(npos)
    conf_loss = (jnp.sum(neg) + pos_sum) / npt
    loc_loss = _ALPHA * jnp.sum(loc_ref[...]) / (npt * 4.0)
    cl_ref[...] = conf_loss.reshape(1, 1)
    ll_ref[...] = loc_loss.reshape(1, 1)
    tot_ref[...] = (conf_loss + loc_loss).reshape(1, 1)


@jax.jit
def kernel(pred_cls, pred_loc, gt_boxes, gt_labels, center_anchor):
    B, P, C = pred_cls.shape
    O = gt_boxes.shape[1]
    f32 = jnp.float32

    caT = center_anchor.T
    plT = jnp.transpose(pred_loc, (0, 2, 1))
    gl3 = gt_labels.reshape(B, O, 1)

    tc3, pos3, locp = pl.pallas_call(
        _match_kernel,
        grid=(B,),
        in_specs=[
            pl.BlockSpec((1, O, 4), lambda b: (b, 0, 0)),
            pl.BlockSpec((1, O, 1), lambda b: (b, 0, 0)),
            pl.BlockSpec((4, P), lambda b: (0, 0)),
            pl.BlockSpec((1, 4, P), lambda b: (b, 0, 0)),
        ],
        out_specs=[
            pl.BlockSpec((1, 1, P), lambda b: (b, 0, 0)),
            pl.BlockSpec((1, 1, P), lambda b: (b, 0, 0)),
            pl.BlockSpec((1, 1, 1), lambda b: (b, 0, 0)),
        ],
        out_shape=[
            jax.ShapeDtypeStruct((B, 1, P), jnp.int32),
            jax.ShapeDtypeStruct((B, 1, P), f32),
            jax.ShapeDtypeStruct((B, 1, 1), f32),
        ],
    )(gt_boxes, gl3, caT, plT)

    tc_col = tc3.reshape(B, P, 1)
    conf3 = pl.pallas_call(
        _ce_kernel,
        grid=(B, pl.cdiv(P, _PB)),
        in_specs=[
            pl.BlockSpec((1, _PB, C), lambda b, p: (b, p, 0)),
            pl.BlockSpec((1, _PB, 1), lambda b, p: (b, p, 0)),
        ],
        out_specs=pl.BlockSpec((1, _PB, 1), lambda b, p: (b, p, 0)),
        out_shape=jax.ShapeDtypeStruct((B, P, 1), f32),
    )(pred_cls, tc_col)

    conf2 = conf3.reshape(B, P)
    pos2 = pos3.reshape(B, P)
    locp2 = locp.reshape(B, 1)
    tot, cl, ll = pl.pallas_call(
        _mine_kernel,
        grid=(1,),
        in_specs=[
            pl.BlockSpec((B, P), lambda i: (0, 0)),
            pl.BlockSpec((B, P), lambda i: (0, 0)),
            pl.BlockSpec((B, 1), lambda i: (0, 0)),
        ],
        out_specs=[
            pl.BlockSpec((1, 1), lambda i: (0, 0)),
            pl.BlockSpec((1, 1), lambda i: (0, 0)),
            pl.BlockSpec((1, 1), lambda i: (0, 0)),
        ],
        out_shape=[
            jax.ShapeDtypeStruct((1, 1), f32),
            jax.ShapeDtypeStruct((1, 1), f32),
            jax.ShapeDtypeStruct((1, 1), f32),
        ],
    )(conf2, pos2, locp2)

    return (tot[0, 0], (cl[0, 0], ll[0, 0]))


# speedup vs baseline: 24.3578x; 2.1406x over previous
import jax
import jax.numpy as jnp
from jax.experimental import pallas as pl

_THRESHOLD = 0.5
_NEG_POS_RATIO = 3
_ALPHA = 10.0

_PB = 1024


def _match_kernel(gb_ref, gl_ref, caT_ref, plT_ref, tc_ref, pos_ref, loc_ref):
    O = gb_ref.shape[1]
    P = caT_ref.shape[1]
    f32 = jnp.float32

    gb = gb_ref[0]
    bx1 = gb[:, 0:1]
    by1 = gb[:, 1:2]
    bx2 = gb[:, 2:3]
    by2 = gb[:, 3:4]
    ca = caT_ref[...]
    acx = ca[0:1, :]
    acy = ca[1:2, :]
    aw = ca[2:3, :]
    ah = ca[3:4, :]
    ax1 = acx - aw / 2.0
    ay1 = acy - ah / 2.0
    ax2 = acx + aw / 2.0
    ay2 = acy + ah / 2.0
    a2 = (ax2 - ax1) * (ay2 - ay1)
    a1 = (bx2 - bx1) * (by2 - by1)

    lox = jnp.maximum(bx1, ax1)
    loy = jnp.maximum(by1, ay1)
    hix = jnp.minimum(bx2, ax2)
    hiy = jnp.minimum(by2, ay2)
    w = jnp.maximum(hix - lox, 0.0)
    h = jnp.maximum(hiy - loy, 0.0)
    inter = w * h
    iou = inter / (a1 + a2 - inter)

    iota_p = jax.lax.broadcasted_iota(jnp.int32, (O, P), 1)
    rmax = jnp.max(iou, axis=1, keepdims=True)
    ridx = jnp.min(jnp.where(iou == rmax, iota_p, P), axis=1, keepdims=True)
    iou_mod = jnp.where(iota_p == ridx, 1.0, iou)

    cmax = jnp.max(iou_mod, axis=0, keepdims=True)
    posf = (cmax >= _THRESHOLD).astype(f32)
    iota_o = jax.lax.broadcasted_iota(jnp.int32, (O, P), 0)
    bo = jnp.min(jnp.where(iou_mod == cmax, iota_o, O), axis=0, keepdims=True)
    onehot = (iota_o == bo).astype(f32)

    lab = gl_ref[0].astype(f32)
    tcls = jnp.sum(onehot * (lab + 1.0), axis=0, keepdims=True)
    tc_ref[...] = (tcls * posf).astype(jnp.int32).reshape(1, 1, P)
    pos_ref[...] = posf.reshape(1, 1, P)

    gx1 = jnp.sum(onehot * bx1, axis=0, keepdims=True)
    gy1 = jnp.sum(onehot * by1, axis=0, keepdims=True)
    gx2 = jnp.sum(onehot * bx2, axis=0, keepdims=True)
    gy2 = jnp.sum(onehot * by2, axis=0, keepdims=True)
    gcx = (gx1 + gx2) / 2.0
    gcy = (gy1 + gy2) / 2.0
    gw = gx2 - gx1
    gh = gy2 - gy1
    t0 = (gcx - acx) / (aw / 10.0)
    t1 = (gcy - acy) / (ah / 10.0)
    t2 = jnp.log(gw / aw) * 5.0
    t3 = jnp.log(gh / ah) * 5.0

    pq = plT_ref[0]
    s = (jnp.abs(pq[0:1, :] - t0) + jnp.abs(pq[1:2, :] - t1)
         + jnp.abs(pq[2:3, :] - t2) + jnp.abs(pq[3:4, :] - t3))
    loc_ref[...] = jnp.sum(s * posf).reshape(1, 1, 1)


def _ce_kernel(x_ref, tc_ref, out_ref):
    P, C = x_ref.shape[1], x_ref.shape[2]
    i = pl.program_id(0) % 8
    x = x_ref[0]
    xt = jnp.transpose(x, (1, 0))
    e = jnp.exp(xt)
    s = jnp.sum(e, axis=0, keepdims=True)
    tc = tc_ref[pl.ds(i, 1), :]
    iota_c = jax.lax.broadcasted_iota(jnp.int32, (C, P), 0)
    xtru = jnp.sum(jnp.where(iota_c == tc, xt, 0.0), axis=0, keepdims=True)
    out_ref[pl.ds(i, 1), :] = jnp.log(s) - xtru


def _mine_kernel(conf_ref, pos_ref, loc_ref, tot_ref, cl_ref, ll_ref):
    f32 = jnp.float32
    i32 = jnp.int32
    conf = conf_ref[...]
    pos = pos_ref[...]
    npos = jnp.sum(pos, axis=1, keepdims=True)
    K = (npos * float(_NEG_POS_RATIO)).astype(i32)
    v = conf * (1.0 - pos)
    u = jax.lax.bitcast_convert_type(v, i32)

    T = jnp.zeros(K.shape, i32)
    for b in range(30, -1, -1):
        cand = T | jnp.int32(1 << b)
        cnt = jnp.sum((u >= cand).astype(i32), axis=1, keepdims=True)
        T = jnp.where(cnt >= K, cand, T)
    Tf = jax.lax.bitcast_convert_type(T, f32)

    gt = v > Tf
    cntgt = jnp.sum(gt.astype(f32), axis=1, keepdims=True)
    sumgt = jnp.sum(jnp.where(gt, v, 0.0), axis=1, keepdims=True)
    neg = sumgt + (K.astype(f32) - cntgt) * Tf

    pos_sum = jnp.sum(conf * pos)
    npt = jnp.sum(npos)
    conf_loss = (jnp.sum(neg) + pos_sum) / npt
    loc_loss = _ALPHA * jnp.sum(loc_ref[...]) / (npt * 4.0)
    cl_ref[...] = conf_loss.reshape(1, 1)
    ll_ref[...] = loc_loss.reshape(1, 1)
    tot_ref[...] = (conf_loss + loc_loss).reshape(1, 1)


@jax.jit
def kernel(pred_cls, pred_loc, gt_boxes, gt_labels, center_anchor):
    B, P, C = pred_cls.shape
    O = gt_boxes.shape[1]
    f32 = jnp.float32

    caT = center_anchor.T
    plT = jnp.transpose(pred_loc, (0, 2, 1))
    gl3 = gt_labels.reshape(B, O, 1)

    tc3, pos3, locp = pl.pallas_call(
        _match_kernel,
        grid=(B,),
        in_specs=[
            pl.BlockSpec((1, O, 4), lambda b: (b, 0, 0)),
            pl.BlockSpec((1, O, 1), lambda b: (b, 0, 0)),
            pl.BlockSpec((4, P), lambda b: (0, 0)),
            pl.BlockSpec((1, 4, P), lambda b: (b, 0, 0)),
        ],
        out_specs=[
            pl.BlockSpec((1, 1, P), lambda b: (b, 0, 0)),
            pl.BlockSpec((1, 1, P), lambda b: (b, 0, 0)),
            pl.BlockSpec((1, 1, 1), lambda b: (b, 0, 0)),
        ],
        out_shape=[
            jax.ShapeDtypeStruct((B, 1, P), jnp.int32),
            jax.ShapeDtypeStruct((B, 1, P), f32),
            jax.ShapeDtypeStruct((B, 1, 1), f32),
        ],
    )(gt_boxes, gl3, caT, plT)

    tc2 = tc3.reshape(B, P)
    conf2 = pl.pallas_call(
        _ce_kernel,
        grid=(B,),
        in_specs=[
            pl.BlockSpec((1, P, C), lambda b: (b, 0, 0)),
            pl.BlockSpec((8, P), lambda b: (b // 8, 0)),
        ],
        out_specs=pl.BlockSpec((8, P), lambda b: (b // 8, 0)),
        out_shape=jax.ShapeDtypeStruct((B, P), f32),
    )(pred_cls, tc2)

    pos2 = pos3.reshape(B, P)
    locp2 = locp.reshape(B, 1)
    tot, cl, ll = pl.pallas_call(
        _mine_kernel,
        grid=(1,),
        in_specs=[
            pl.BlockSpec((B, P), lambda i: (0, 0)),
            pl.BlockSpec((B, P), lambda i: (0, 0)),
            pl.BlockSpec((B, 1), lambda i: (0, 0)),
        ],
        out_specs=[
            pl.BlockSpec((1, 1), lambda i: (0, 0)),
            pl.BlockSpec((1, 1), lambda i: (0, 0)),
            pl.BlockSpec((1, 1), lambda i: (0, 0)),
        ],
        out_shape=[
            jax.ShapeDtypeStruct((1, 1), f32),
            jax.ShapeDtypeStruct((1, 1), f32),
            jax.ShapeDtypeStruct((1, 1), f32),
        ],
    )(conf2, pos2, locp2)

    return (tot[0, 0], (cl[0, 0], ll[0, 0]))


# speedup vs baseline: 26.3409x; 1.0814x over previous
import jax
import jax.numpy as jnp
from jax.experimental import pallas as pl

_THRESHOLD = 0.5
_NEG_POS_RATIO = 3
_ALPHA = 10.0

_PB = 1024


def _match_kernel(gb_ref, gl_ref, caT_ref, plT_ref, tc_ref, pos_ref, loc_ref):
    G, O = gb_ref.shape[0], gb_ref.shape[1]
    P = caT_ref.shape[1]
    GO = G * O
    f32 = jnp.float32

    gb = gb_ref[...].reshape(GO, 4)
    bx1 = gb[:, 0:1]
    by1 = gb[:, 1:2]
    bx2 = gb[:, 2:3]
    by2 = gb[:, 3:4]
    ca = caT_ref[...]
    acx = ca[0:1, :]
    acy = ca[1:2, :]
    aw = ca[2:3, :]
    ah = ca[3:4, :]
    ax1 = acx - aw / 2.0
    ay1 = acy - ah / 2.0
    ax2 = acx + aw / 2.0
    ay2 = acy + ah / 2.0
    a2 = (ax2 - ax1) * (ay2 - ay1)
    a1 = (bx2 - bx1) * (by2 - by1)

    lox = jnp.maximum(bx1, ax1)
    loy = jnp.maximum(by1, ay1)
    hix = jnp.minimum(bx2, ax2)
    hiy = jnp.minimum(by2, ay2)
    w = jnp.maximum(hix - lox, 0.0)
    h = jnp.maximum(hiy - loy, 0.0)
    inter = w * h
    iou = inter / (a1 + a2 - inter)

    iota_p = jax.lax.broadcasted_iota(jnp.int32, (GO, P), 1)
    rmax = jnp.max(iou, axis=1, keepdims=True)
    ridx = jnp.min(jnp.where(iou == rmax, iota_p, P), axis=1, keepdims=True)
    iou3 = jnp.where(iota_p == ridx, 1.0, iou).reshape(G, O, P)

    cmax = jnp.max(iou3, axis=1, keepdims=True)
    posf = (cmax >= _THRESHOLD).astype(f32).reshape(G, P)
    iota_o = jax.lax.broadcasted_iota(jnp.int32, (G, O, P), 1)
    bo = jnp.min(jnp.where(iou3 == cmax, iota_o, O), axis=1, keepdims=True)
    onehot = (iota_o == bo).astype(f32)

    lab = gl_ref[...].astype(f32)
    tcls = jnp.sum(onehot * (lab + 1.0), axis=1)
    tc_ref[...] = (tcls * posf).astype(jnp.int32)
    pos_ref[...] = posf

    gb3 = gb.reshape(G, O, 4)
    gx1 = jnp.sum(onehot * gb3[:, :, 0:1], axis=1)
    gy1 = jnp.sum(onehot * gb3[:, :, 1:2], axis=1)
    gx2 = jnp.sum(onehot * gb3[:, :, 2:3], axis=1)
    gy2 = jnp.sum(onehot * gb3[:, :, 3:4], axis=1)
    gcx = (gx1 + gx2) / 2.0
    gcy = (gy1 + gy2) / 2.0
    gw = gx2 - gx1
    gh = gy2 - gy1
    t0 = (gcx - acx) / (aw / 10.0)
    t1 = (gcy - acy) / (ah / 10.0)
    t2 = jnp.log(gw / aw) * 5.0
    t3 = jnp.log(gh / ah) * 5.0

    pq = plT_ref[...]
    s = (jnp.abs(pq[:, 0, :] - t0) + jnp.abs(pq[:, 1, :] - t1)
         + jnp.abs(pq[:, 2, :] - t2) + jnp.abs(pq[:, 3, :] - t3))
    loc_ref[...] = jnp.sum(s * posf, axis=1, keepdims=True)


def _ce_kernel(x_ref, tc_ref, out_ref):
    P, C = x_ref.shape[1], x_ref.shape[2]
    i = pl.program_id(0) % 8
    x = x_ref[0]
    xt = jnp.transpose(x, (1, 0))
    e = jnp.exp(xt)
    s = jnp.sum(e, axis=0, keepdims=True)
    tc = tc_ref[pl.ds(i, 1), :]
    iota_c = jax.lax.broadcasted_iota(jnp.int32, (C, P), 0)
    xtru = jnp.sum(jnp.where(iota_c == tc, xt, 0.0), axis=0, keepdims=True)
    out_ref[pl.ds(i, 1), :] = jnp.log(s) - xtru


def _mine_kernel(conf_ref, pos_ref, loc_ref, tot_ref, cl_ref, ll_ref):
    f32 = jnp.float32
    i32 = jnp.int32
    conf = conf_ref[...]
    pos = pos_ref[...]
    npos = jnp.sum(pos, axis=1, keepdims=True)
    K = (npos * float(_NEG_POS_RATIO)).astype(i32)
    v = conf * (1.0 - pos)
    u = jax.lax.bitcast_convert_type(v, i32)

    T = jnp.zeros(K.shape, i32)
    for b in range(30, -1, -1):
        cand = T | jnp.int32(1 << b)
        cnt = jnp.sum((u >= cand).astype(i32), axis=1, keepdims=True)
        T = jnp.where(cnt >= K, cand, T)
    Tf = jax.lax.bitcast_convert_type(T, f32)

    gt = v > Tf
    cntgt = jnp.sum(gt.astype(f32), axis=1, keepdims=True)
    sumgt = jnp.sum(jnp.where(gt, v, 0.0), axis=1, keepdims=True)
    neg = sumgt + (K.astype(f32) - cntgt) * Tf

    pos_sum = jnp.sum(conf * pos)
    npt = jnp.sum(npos)
    conf_loss = (jnp.sum(neg) + pos_sum) / npt
    loc_loss = _ALPHA * jnp.sum(loc_ref[...]) / (npt * 4.0)
    cl_ref[...] = conf_loss.reshape(1, 1)
    ll_ref[...] = loc_loss.reshape(1, 1)
    tot_ref[...] = (conf_loss + loc_loss).reshape(1, 1)


@jax.jit
def kernel(pred_cls, pred_loc, gt_boxes, gt_labels, center_anchor):
    B, P, C = pred_cls.shape
    O = gt_boxes.shape[1]
    f32 = jnp.float32

    caT = center_anchor.T
    plT = jnp.transpose(pred_loc, (0, 2, 1))
    gl3 = gt_labels.reshape(B, O, 1)

    G = 8
    tc2, pos2, locp = pl.pallas_call(
        _match_kernel,
        grid=(B // G,),
        in_specs=[
            pl.BlockSpec((G, O, 4), lambda b: (b, 0, 0)),
            pl.BlockSpec((G, O, 1), lambda b: (b, 0, 0)),
            pl.BlockSpec((4, P), lambda b: (0, 0)),
            pl.BlockSpec((G, 4, P), lambda b: (b, 0, 0)),
        ],
        out_specs=[
            pl.BlockSpec((G, P), lambda b: (b, 0)),
            pl.BlockSpec((G, P), lambda b: (b, 0)),
            pl.BlockSpec((G, 1), lambda b: (b, 0)),
        ],
        out_shape=[
            jax.ShapeDtypeStruct((B, P), jnp.int32),
            jax.ShapeDtypeStruct((B, P), f32),
            jax.ShapeDtypeStruct((B, 1), f32),
        ],
    )(gt_boxes, gl3, caT, plT)
    conf2 = pl.pallas_call(
        _ce_kernel,
        grid=(B,),
        in_specs=[
            pl.BlockSpec((1, P, C), lambda b: (b, 0, 0)),
            pl.BlockSpec((8, P), lambda b: (b // 8, 0)),
        ],
        out_specs=pl.BlockSpec((8, P), lambda b: (b // 8, 0)),
        out_shape=jax.ShapeDtypeStruct((B, P), f32),
    )(pred_cls, tc2)

    tot, cl, ll = pl.pallas_call(
        _mine_kernel,
        grid=(1,),
        in_specs=[
            pl.BlockSpec((B, P), lambda i: (0, 0)),
            pl.BlockSpec((B, P), lambda i: (0, 0)),
            pl.BlockSpec((B, 1), lambda i: (0, 0)),
        ],
        out_specs=[
            pl.BlockSpec((1, 1), lambda i: (0, 0)),
            pl.BlockSpec((1, 1), lambda i: (0, 0)),
            pl.BlockSpec((1, 1), lambda i: (0, 0)),
        ],
        out_shape=[
            jax.ShapeDtypeStruct((1, 1), f32),
            jax.ShapeDtypeStruct((1, 1), f32),
            jax.ShapeDtypeStruct((1, 1), f32),
        ],
    )(conf2, pos2, locp)

    return (tot[0, 0], (cl[0, 0], ll[0, 0]))


# speedup vs baseline: 26.8896x; 1.0208x over previous
import jax
import jax.numpy as jnp
from jax.experimental import pallas as pl

_THRESHOLD = 0.5
_NEG_POS_RATIO = 3
_ALPHA = 10.0


def _fused_kernel(x_ref, gb_ref, gl_ref, caT_ref, plT_ref,
                  conf_ref, pos_ref, loc_ref):
    O = gb_ref.shape[1]
    P = caT_ref.shape[1]
    C = x_ref.shape[2]
    f32 = jnp.float32
    i = pl.program_id(0) % 8

    gb = gb_ref[0]
    bx1 = gb[:, 0:1]
    by1 = gb[:, 1:2]
    bx2 = gb[:, 2:3]
    by2 = gb[:, 3:4]
    ca = caT_ref[...]
    acx = ca[0:1, :]
    acy = ca[1:2, :]
    aw = ca[2:3, :]
    ah = ca[3:4, :]
    ax1 = acx - aw / 2.0
    ay1 = acy - ah / 2.0
    ax2 = acx + aw / 2.0
    ay2 = acy + ah / 2.0
    a2 = (ax2 - ax1) * (ay2 - ay1)
    a1 = (bx2 - bx1) * (by2 - by1)

    lox = jnp.maximum(bx1, ax1)
    loy = jnp.maximum(by1, ay1)
    hix = jnp.minimum(bx2, ax2)
    hiy = jnp.minimum(by2, ay2)
    w = jnp.maximum(hix - lox, 0.0)
    h = jnp.maximum(hiy - loy, 0.0)
    inter = w * h
    iou = inter / (a1 + a2 - inter)

    iota_p = jax.lax.broadcasted_iota(jnp.int32, (O, P), 1)
    rmax = jnp.max(iou, axis=1, keepdims=True)
    ridx = jnp.min(jnp.where(iou == rmax, iota_p, P), axis=1, keepdims=True)
    iou_mod = jnp.where(iota_p == ridx, 1.0, iou)

    cmax = jnp.max(iou_mod, axis=0, keepdims=True)
    posf = (cmax >= _THRESHOLD).astype(f32)
    iota_o = jax.lax.broadcasted_iota(jnp.int32, (O, P), 0)
    bo = jnp.min(jnp.where(iou_mod == cmax, iota_o, O), axis=0, keepdims=True)
    onehot = (iota_o == bo).astype(f32)

    lab = gl_ref[0].astype(f32)
    tclsf = jnp.sum(onehot * (lab + 1.0), axis=0, keepdims=True)
    tc = (tclsf * posf).astype(jnp.int32)
    pos_ref[pl.ds(i, 1), :] = posf

    gx1 = jnp.sum(onehot * bx1, axis=0, keepdims=True)
    gy1 = jnp.sum(onehot * by1, axis=0, keepdims=True)
    gx2 = jnp.sum(onehot * bx2, axis=0, keepdims=True)
    gy2 = jnp.sum(onehot * by2, axis=0, keepdims=True)
    gcx = (gx1 + gx2) / 2.0
    gcy = (gy1 + gy2) / 2.0
    gw = gx2 - gx1
    gh = gy2 - gy1
    t0 = (gcx - acx) / (aw / 10.0)
    t1 = (gcy - acy) / (ah / 10.0)
    t2 = jnp.log(gw / aw) * 5.0
    t3 = jnp.log(gh / ah) * 5.0

    pq = plT_ref[0]
    s4 = (jnp.abs(pq[0:1, :] - t0) + jnp.abs(pq[1:2, :] - t1)
          + jnp.abs(pq[2:3, :] - t2) + jnp.abs(pq[3:4, :] - t3))
    loc_ref[pl.ds(i, 1), :] = jnp.sum(s4 * posf).reshape(1, 1)

    x = x_ref[0]
    xt = jnp.transpose(x, (1, 0))
    e = jnp.exp(xt)
    se = jnp.sum(e, axis=0, keepdims=True)
    iota_c = jax.lax.broadcasted_iota(jnp.int32, (C, P), 0)
    xtru = jnp.sum(jnp.where(iota_c == tc, xt, 0.0), axis=0, keepdims=True)
    conf_ref[pl.ds(i, 1), :] = jnp.log(se) - xtru


def _mine_kernel(conf_ref, pos_ref, loc_ref, tot_ref, cl_ref, ll_ref):
    f32 = jnp.float32
    i32 = jnp.int32
    conf = conf_ref[...]
    pos = pos_ref[...]
    npos = jnp.sum(pos, axis=1, keepdims=True)
    K = (npos * float(_NEG_POS_RATIO)).astype(i32)
    v = conf * (1.0 - pos)
    u = jax.lax.bitcast_convert_type(v, i32)

    T = jnp.zeros(K.shape, i32)
    for b in range(30, -1, -1):
        cand = T | jnp.int32(1 << b)
        cnt = jnp.sum((u >= cand).astype(i32), axis=1, keepdims=True)
        T = jnp.where(cnt >= K, cand, T)
    Tf = jax.lax.bitcast_convert_type(T, f32)

    gt = v > Tf
    cntgt = jnp.sum(gt.astype(f32), axis=1, keepdims=True)
    sumgt = jnp.sum(jnp.where(gt, v, 0.0), axis=1, keepdims=True)
    neg = sumgt + (K.astype(f32) - cntgt) * Tf

    pos_sum = jnp.sum(conf * pos)
    npt = jnp.sum(npos)
    conf_loss = (jnp.sum(neg) + pos_sum) / npt
    loc_loss = _ALPHA * jnp.sum(loc_ref[...]) / (npt * 4.0)
    cl_ref[...] = conf_loss.reshape(1, 1)
    ll_ref[...] = loc_loss.reshape(1, 1)
    tot_ref[...] = (conf_loss + loc_loss).reshape(1, 1)


@jax.jit
def kernel(pred_cls, pred_loc, gt_boxes, gt_labels, center_anchor):
    B, P, C = pred_cls.shape
    O = gt_boxes.shape[1]
    f32 = jnp.float32

    caT = center_anchor.T
    plT = jnp.transpose(pred_loc, (0, 2, 1))
    gl3 = gt_labels.reshape(B, O, 1)

    conf2, pos2, locp = pl.pallas_call(
        _fused_kernel,
        grid=(B,),
        in_specs=[
            pl.BlockSpec((1, P, C), lambda b: (b, 0, 0)),
            pl.BlockSpec((1, O, 4), lambda b: (b, 0, 0)),
            pl.BlockSpec((1, O, 1), lambda b: (b, 0, 0)),
            pl.BlockSpec((4, P), lambda b: (0, 0)),
            pl.BlockSpec((1, 4, P), lambda b: (b, 0, 0)),
        ],
        out_specs=[
            pl.BlockSpec((8, P), lambda b: (b // 8, 0)),
            pl.BlockSpec((8, P), lambda b: (b // 8, 0)),
            pl.BlockSpec((8, 1), lambda b: (b // 8, 0)),
        ],
        out_shape=[
            jax.ShapeDtypeStruct((B, P), f32),
            jax.ShapeDtypeStruct((B, P), f32),
            jax.ShapeDtypeStruct((B, 1), f32),
        ],
    )(pred_cls, gt_boxes, gl3, caT, plT)

    tot, cl, ll = pl.pallas_call(
        _mine_kernel,
        grid=(1,),
        in_specs=[
            pl.BlockSpec((B, P), lambda i: (0, 0)),
            pl.BlockSpec((B, P), lambda i: (0, 0)),
            pl.BlockSpec((B, 1), lambda i: (0, 0)),
        ],
        out_specs=[
            pl.BlockSpec((1, 1), lambda i: (0, 0)),
            pl.BlockSpec((1, 1), lambda i: (0, 0)),
            pl.BlockSpec((1, 1), lambda i: (0, 0)),
        ],
        out_shape=[
            jax.ShapeDtypeStruct((1, 1), f32),
            jax.ShapeDtypeStruct((1, 1), f32),
            jax.ShapeDtypeStruct((1, 1), f32),
        ],
    )(conf2, pos2, locp)

    return (tot[0, 0], (cl[0, 0], ll[0, 0]))
